# Initial kernel scaffold; baseline (speedup 1.0000x reference)
import jax
import jax.numpy as jnp
from jax.experimental import pallas as pl


def kernel(pos_ids_0, pos_ids_1, pos_ids_2, table_0, table_1, table_2):
    raise NotImplementedError("write your pallas kernel here")


# speedup vs baseline: 14.6801x; 14.6801x over previous
import functools
import jax
import jax.numpy as jnp
from jax import lax
from jax.experimental import pallas as pl
from jax.experimental.pallas import tpu as pltpu
from jax.experimental.pallas import tpu_sc as plsc

B, L = 4096, 200
N = B * L
D0, D1, D2 = 64, 32, 32
DO = D0 + D1 + D2
NC, NS = 2, 16
NW = NC * NS
C = 128
TOK_PER_W = N // NW
CHUNKS_PER_W = TOK_PER_W // C


def _encoder_kernel(ids0, ids1, ids2, t0, t1, t2, out,
                    t0_s, t1_s, t2_s,
                    idx0_v, idx1_v, idx2_v,
                    e0_v, e1_v, e2_v,
                    sem0, sem1, sem2):
    cid = lax.axis_index("c")
    sid = lax.axis_index("s")
    wid = sid * NC + cid

    @pl.when(sid == 0)
    def _stage():
        pltpu.sync_copy(t0, t0_s)
        pltpu.sync_copy(t1, t1_s)
        pltpu.sync_copy(t2, t2_s)

    plsc.subcore_barrier()

    row0 = wid * CHUNKS_PER_W

    def body(i, carry):
        r = row0 + i
        pltpu.sync_copy(ids0.at[r], idx0_v)
        pltpu.sync_copy(ids1.at[r], idx1_v)
        pltpu.sync_copy(ids2.at[r], idx2_v)
        cp0 = pltpu.async_copy(t0_s.at[idx0_v], e0_v, sem0)
        cp1 = pltpu.async_copy(t1_s.at[idx1_v], e1_v, sem1)
        cp2 = pltpu.async_copy(t2_s.at[idx2_v], e2_v, sem2)
        cp0.wait()
        cp1.wait()
        cp2.wait()
        base = r * C
        pltpu.sync_copy(e0_v, out.at[pl.ds(base, C), pl.ds(0, D0)])
        pltpu.sync_copy(e1_v, out.at[pl.ds(base, C), pl.ds(D0, D1)])
        pltpu.sync_copy(e2_v, out.at[pl.ds(base, C), pl.ds(D0 + D1, D2)])
        return carry

    lax.fori_loop(0, CHUNKS_PER_W, body, 0)


def kernel(pos_ids_0, pos_ids_1, pos_ids_2, table_0, table_1, table_2):
    ids0 = pos_ids_0.reshape(N // C, C)
    ids1 = pos_ids_1.reshape(N // C, C)
    ids2 = pos_ids_2.reshape(N // C, C)

    mesh = plsc.VectorSubcoreMesh(core_axis_name="c", subcore_axis_name="s")
    run = pl.kernel(
        _encoder_kernel,
        out_type=jax.ShapeDtypeStruct((N, DO), jnp.float32),
        mesh=mesh,
        compiler_params=pltpu.CompilerParams(use_tc_tiling_on_sc=False),
        scratch_types=[
            pltpu.VMEM_SHARED((2048, D0), jnp.float32),
            pltpu.VMEM_SHARED((2048, D1), jnp.float32),
            pltpu.VMEM_SHARED((512, D2), jnp.float32),
            pltpu.VMEM((C,), jnp.int32),
            pltpu.VMEM((C,), jnp.int32),
            pltpu.VMEM((C,), jnp.int32),
            pltpu.VMEM((C, D0), jnp.float32),
            pltpu.VMEM((C, D1), jnp.float32),
            pltpu.VMEM((C, D2), jnp.float32),
            pltpu.SemaphoreType.DMA,
            pltpu.SemaphoreType.DMA,
            pltpu.SemaphoreType.DMA,
        ],
    )
    out = run(ids0, ids1, ids2, table_0, table_1, table_2)
    return out.reshape(B, L, DO)


# speedup vs baseline: 23.6474x; 1.6108x over previous
import functools
import jax
import jax.numpy as jnp
from jax import lax
from jax.experimental import pallas as pl
from jax.experimental.pallas import tpu as pltpu
from jax.experimental.pallas import tpu_sc as plsc

B, L = 4096, 200
N = B * L
D0, D1, D2 = 64, 32, 32
DO = D0 + D1 + D2
NC, NS = 2, 16
NW = NC * NS
C = 512
TOK_PER_W = N // NW
CHUNKS_PER_W = TOK_PER_W // C


def _encoder_kernel(ids0, ids1, ids2, t0, t1, t2, out,
                    t0_s, t1_s, t2_s,
                    idx0_v, idx1_v, idx2_v,
                    e0_v, e1_v, e2_v,
                    sem0, sem1, sem2):
    cid = lax.axis_index("c")
    sid = lax.axis_index("s")
    wid = sid * NC + cid

    @pl.when(sid == 0)
    def _stage():
        pltpu.sync_copy(t0, t0_s)
        pltpu.sync_copy(t1, t1_s)
        pltpu.sync_copy(t2, t2_s)

    plsc.subcore_barrier()

    row0 = wid * CHUNKS_PER_W

    def body(i, carry):
        r = row0 + i
        pltpu.sync_copy(ids0.at[r], idx0_v)
        pltpu.sync_copy(ids1.at[r], idx1_v)
        pltpu.sync_copy(ids2.at[r], idx2_v)
        cp0 = pltpu.async_copy(t0_s.at[idx0_v], e0_v, sem0)
        cp1 = pltpu.async_copy(t1_s.at[idx1_v], e1_v, sem1)
        cp2 = pltpu.async_copy(t2_s.at[idx2_v], e2_v, sem2)
        cp0.wait()
        cp1.wait()
        cp2.wait()
        base = r * C
        pltpu.sync_copy(e0_v, out.at[pl.ds(base, C), pl.ds(0, D0)])
        pltpu.sync_copy(e1_v, out.at[pl.ds(base, C), pl.ds(D0, D1)])
        pltpu.sync_copy(e2_v, out.at[pl.ds(base, C), pl.ds(D0 + D1, D2)])
        return carry

    lax.fori_loop(0, CHUNKS_PER_W, body, 0)


def kernel(pos_ids_0, pos_ids_1, pos_ids_2, table_0, table_1, table_2):
    ids0 = pos_ids_0.reshape(N // C, C)
    ids1 = pos_ids_1.reshape(N // C, C)
    ids2 = pos_ids_2.reshape(N // C, C)

    mesh = plsc.VectorSubcoreMesh(core_axis_name="c", subcore_axis_name="s")
    run = pl.kernel(
        _encoder_kernel,
        out_type=jax.ShapeDtypeStruct((N, DO), jnp.float32),
        mesh=mesh,
        compiler_params=pltpu.CompilerParams(use_tc_tiling_on_sc=False),
        scratch_types=[
            pltpu.VMEM_SHARED((2048, D0), jnp.float32),
            pltpu.VMEM_SHARED((2048, D1), jnp.float32),
            pltpu.VMEM_SHARED((512, D2), jnp.float32),
            pltpu.VMEM((C,), jnp.int32),
            pltpu.VMEM((C,), jnp.int32),
            pltpu.VMEM((C,), jnp.int32),
            pltpu.VMEM((C, D0), jnp.float32),
            pltpu.VMEM((C, D1), jnp.float32),
            pltpu.VMEM((C, D2), jnp.float32),
            pltpu.SemaphoreType.DMA,
            pltpu.SemaphoreType.DMA,
            pltpu.SemaphoreType.DMA,
        ],
    )
    out = run(ids0, ids1, ids2, table_0, table_1, table_2)
    return out.reshape(B, L, DO)


# speedup vs baseline: 33.3239x; 1.4092x over previous
import jax
import jax.numpy as jnp
from jax import lax
from jax.experimental import pallas as pl
from jax.experimental.pallas import tpu as pltpu
from jax.experimental.pallas import tpu_sc as plsc

B, L = 4096, 200
N = B * L
D0, D1, D2 = 64, 32, 32
DO = D0 + D1 + D2
NC, NS = 2, 16
NW = NC * NS
C = 256
TOK_PER_W = N // NW
CHUNKS_PER_W = TOK_PER_W // C


def _encoder_kernel(ids0, ids1, ids2, t0, t1, t2, out,
                    t0_s, t1_s, t2_s,
                    idx0_v, idx1_v, idx2_v,
                    e0_v, e1_v, e2_v,
                    sem_i, sem_g, sem_w):
    cid = lax.axis_index("c")
    sid = lax.axis_index("s")
    wid = sid * NC + cid

    @pl.when(sid == 0)
    def _stage():
        pltpu.sync_copy(t0, t0_s)
        pltpu.sync_copy(t1, t1_s)
        pltpu.sync_copy(t2, t2_s)

    plsc.subcore_barrier()

    row0 = wid * CHUNKS_PER_W

    def fire_idx_fetch(c, p):
        r = row0 + lax.rem(c, CHUNKS_PER_W)
        pltpu.async_copy(ids0.at[r], idx0_v.at[p], sem_i.at[p])
        pltpu.async_copy(ids1.at[r], idx1_v.at[p], sem_i.at[p])
        pltpu.async_copy(ids2.at[r], idx2_v.at[p], sem_i.at[p])

    def wait_idx_fetch(p):
        pltpu.make_async_copy(ids0.at[0], idx0_v.at[p], sem_i.at[p]).wait()
        pltpu.make_async_copy(ids1.at[0], idx1_v.at[p], sem_i.at[p]).wait()
        pltpu.make_async_copy(ids2.at[0], idx2_v.at[p], sem_i.at[p]).wait()

    def out_slices(base):
        return (out.at[pl.ds(base, C), pl.ds(0, D0)],
                out.at[pl.ds(base, C), pl.ds(D0, D1)],
                out.at[pl.ds(base, C), pl.ds(D0 + D1, D2)])

    def wait_writes(p, e0, e1, e2):
        o0, o1, o2 = out_slices(0)
        pltpu.make_async_copy(e0, o0, sem_w.at[p]).wait()
        pltpu.make_async_copy(e1, o1, sem_w.at[p]).wait()
        pltpu.make_async_copy(e2, o2, sem_w.at[p]).wait()

    def chunk_step(c, p):
        e0, e1, e2 = e0_v.at[p], e1_v.at[p], e2_v.at[p]
        wait_idx_fetch(p)
        @pl.when(c >= 2)
        def _():
            wait_writes(p, e0, e1, e2)
        g0 = pltpu.async_copy(t0_s.at[idx0_v.at[p]], e0, sem_g.at[p])
        g1 = pltpu.async_copy(t1_s.at[idx1_v.at[p]], e1, sem_g.at[p])
        g2 = pltpu.async_copy(t2_s.at[idx2_v.at[p]], e2, sem_g.at[p])
        g0.wait()
        g1.wait()
        g2.wait()
        fire_idx_fetch(c + 1, 1 - p)
        base = (row0 + c) * C
        o0, o1, o2 = out_slices(base)
        pltpu.async_copy(e0, o0, sem_w.at[p])
        pltpu.async_copy(e1, o1, sem_w.at[p])
        pltpu.async_copy(e2, o2, sem_w.at[p])

    fire_idx_fetch(0, 0)

    def body(i, carry):
        chunk_step(2 * i, 0)
        chunk_step(2 * i + 1, 1)
        return carry

    lax.fori_loop(0, CHUNKS_PER_W // 2, body, 0)

    wait_writes(0, e0_v.at[0], e1_v.at[0], e2_v.at[0])
    wait_writes(1, e0_v.at[1], e1_v.at[1], e2_v.at[1])
    wait_idx_fetch(0)


def kernel(pos_ids_0, pos_ids_1, pos_ids_2, table_0, table_1, table_2):
    ids0 = pos_ids_0.reshape(N // C, C)
    ids1 = pos_ids_1.reshape(N // C, C)
    ids2 = pos_ids_2.reshape(N // C, C)

    mesh = plsc.VectorSubcoreMesh(core_axis_name="c", subcore_axis_name="s")
    run = pl.kernel(
        _encoder_kernel,
        out_type=jax.ShapeDtypeStruct((N, DO), jnp.float32),
        mesh=mesh,
        compiler_params=pltpu.CompilerParams(use_tc_tiling_on_sc=False),
        scratch_types=[
            pltpu.VMEM_SHARED((2048, D0), jnp.float32),
            pltpu.VMEM_SHARED((2048, D1), jnp.float32),
            pltpu.VMEM_SHARED((512, D2), jnp.float32),
            pltpu.VMEM((2, C), jnp.int32),
            pltpu.VMEM((2, C), jnp.int32),
            pltpu.VMEM((2, C), jnp.int32),
            pltpu.VMEM((2, C, D0), jnp.float32),
            pltpu.VMEM((2, C, D1), jnp.float32),
            pltpu.VMEM((2, C, D2), jnp.float32),
            pltpu.SemaphoreType.DMA((2,)),
            pltpu.SemaphoreType.DMA((2,)),
            pltpu.SemaphoreType.DMA((2,)),
        ],
    )
    out = run(ids0, ids1, ids2, table_0, table_1, table_2)
    return out.reshape(B, L, DO)


# speedup vs baseline: 41.7719x; 1.2535x over previous
import jax
import jax.numpy as jnp
from jax import lax
from jax.experimental import pallas as pl
from jax.experimental.pallas import tpu as pltpu
from jax.experimental.pallas import tpu_sc as plsc

B, L = 4096, 200
N = B * L
D0, D1, D2 = 64, 32, 32
DO = D0 + D1 + D2
NC, NS = 2, 16
NW = NC * NS
C = 400
TOK_PER_W = N // NW
CHUNKS_PER_W = TOK_PER_W // C


def _encoder_kernel(ids0, ids1, ids2, t0, t1, t2, out,
                    t0_s, t1_s, t2_s,
                    idx0_v, idx1_v, idx2_v,
                    e0_v, e1_v, e2_v,
                    sem_i, sem_g, sem_w):
    cid = lax.axis_index("c")
    sid = lax.axis_index("s")
    wid = sid * NC + cid

    @pl.when(sid == 0)
    def _stage():
        pltpu.sync_copy(t0, t0_s)
        pltpu.sync_copy(t1, t1_s)
        pltpu.sync_copy(t2, t2_s)

    plsc.subcore_barrier()

    row0 = wid * CHUNKS_PER_W

    def fire_idx_fetch(c, p):
        r = row0 + lax.rem(c, CHUNKS_PER_W)
        pltpu.async_copy(ids0.at[r], idx0_v.at[p], sem_i.at[p])
        pltpu.async_copy(ids1.at[r], idx1_v.at[p], sem_i.at[p])
        pltpu.async_copy(ids2.at[r], idx2_v.at[p], sem_i.at[p])

    def wait_idx_fetch(p):
        pltpu.make_async_copy(ids0.at[0], idx0_v.at[p], sem_i.at[p]).wait()
        pltpu.make_async_copy(ids1.at[0], idx1_v.at[p], sem_i.at[p]).wait()
        pltpu.make_async_copy(ids2.at[0], idx2_v.at[p], sem_i.at[p]).wait()

    def out_slices(base):
        return (out.at[pl.ds(base, C), pl.ds(0, D0)],
                out.at[pl.ds(base, C), pl.ds(D0, D1)],
                out.at[pl.ds(base, C), pl.ds(D0 + D1, D2)])

    def wait_writes(p, e0, e1, e2):
        o0, o1, o2 = out_slices(0)
        pltpu.make_async_copy(e0, o0, sem_w.at[p]).wait()
        pltpu.make_async_copy(e1, o1, sem_w.at[p]).wait()
        pltpu.make_async_copy(e2, o2, sem_w.at[p]).wait()

    def wait_gathers(p, e0, e1, e2):
        pltpu.make_async_copy(t0_s.at[idx0_v.at[p]], e0, sem_g.at[p]).wait()
        pltpu.make_async_copy(t1_s.at[idx1_v.at[p]], e1, sem_g.at[p]).wait()
        pltpu.make_async_copy(t2_s.at[idx2_v.at[p]], e2, sem_g.at[p]).wait()

    def chunk_step(c, p):
        q = 1 - p
        e0, e1, e2 = e0_v.at[p], e1_v.at[p], e2_v.at[p]
        f0, f1, f2 = e0_v.at[q], e1_v.at[q], e2_v.at[q]
        wait_idx_fetch(p)
        @pl.when(c >= 2)
        def _():
            wait_writes(p, e0, e1, e2)
        pltpu.async_copy(t0_s.at[idx0_v.at[p]], e0, sem_g.at[p])
        pltpu.async_copy(t1_s.at[idx1_v.at[p]], e1, sem_g.at[p])
        pltpu.async_copy(t2_s.at[idx2_v.at[p]], e2, sem_g.at[p])

        @pl.when(c >= 1)
        def _():
            wait_gathers(q, f0, f1, f2)
            fire_idx_fetch(c + 1, q)
            base = (row0 + c - 1) * C
            o0, o1, o2 = out_slices(base)
            pltpu.async_copy(f0, o0, sem_w.at[q])
            pltpu.async_copy(f1, o1, sem_w.at[q])
            pltpu.async_copy(f2, o2, sem_w.at[q])

        @pl.when(c == 0)
        def _():
            fire_idx_fetch(1, q)

    fire_idx_fetch(0, 0)

    def body(i, carry):
        chunk_step(2 * i, 0)
        chunk_step(2 * i + 1, 1)
        return carry

    lax.fori_loop(0, CHUNKS_PER_W // 2, body, 0)

    lastp = 1
    el0, el1, el2 = e0_v.at[lastp], e1_v.at[lastp], e2_v.at[lastp]
    wait_gathers(lastp, el0, el1, el2)
    base = (row0 + CHUNKS_PER_W - 1) * C
    o0, o1, o2 = out_slices(base)
    pltpu.async_copy(el0, o0, sem_w.at[lastp])
    pltpu.async_copy(el1, o1, sem_w.at[lastp])
    pltpu.async_copy(el2, o2, sem_w.at[lastp])
    wait_writes(0, e0_v.at[0], e1_v.at[0], e2_v.at[0])
    wait_writes(1, el0, el1, el2)
    wait_idx_fetch(0)


def kernel(pos_ids_0, pos_ids_1, pos_ids_2, table_0, table_1, table_2):
    ids0 = pos_ids_0.reshape(N // C, C)
    ids1 = pos_ids_1.reshape(N // C, C)
    ids2 = pos_ids_2.reshape(N // C, C)

    mesh = plsc.VectorSubcoreMesh(core_axis_name="c", subcore_axis_name="s")
    run = pl.kernel(
        _encoder_kernel,
        out_type=jax.ShapeDtypeStruct((N, DO), jnp.float32),
        mesh=mesh,
        compiler_params=pltpu.CompilerParams(use_tc_tiling_on_sc=False),
        scratch_types=[
            pltpu.VMEM_SHARED((2048, D0), jnp.float32),
            pltpu.VMEM_SHARED((2048, D1), jnp.float32),
            pltpu.VMEM_SHARED((512, D2), jnp.float32),
            pltpu.VMEM((2, C), jnp.int32),
            pltpu.VMEM((2, C), jnp.int32),
            pltpu.VMEM((2, C), jnp.int32),
            pltpu.VMEM((2, C, D0), jnp.float32),
            pltpu.VMEM((2, C, D1), jnp.float32),
            pltpu.VMEM((2, C, D2), jnp.float32),
            pltpu.SemaphoreType.DMA((2,)),
            pltpu.SemaphoreType.DMA((2,)),
            pltpu.SemaphoreType.DMA((2,)),
        ],
    )
    out = run(ids0, ids1, ids2, table_0, table_1, table_2)
    return out.reshape(B, L, DO)
